# baseline (speedup 1.0000x reference)
import functools

import jax
import jax.numpy as jnp
from jax import lax
from jax.experimental import pallas as pl
from jax.experimental.pallas import tpu as pltpu
from jax.experimental.pallas import tpu_sc as plsc

E = 64
TOPK = 2
D = 1024
FF = 768
CAP = 16
ALPHA = 1.702
LIMIT = 7.0
T = 128

NC = 1
NS = 16
L = 16
NW = NC * NS
TPW = T // NW


def _sc_routing(logits_hbm, table_hbm, logits_v, out_v, cnt_v, all_v, shared):
    wid = lax.axis_index("s") * NC + lax.axis_index("c")
    base = wid * TPW
    pltpu.sync_copy(logits_hbm.at[pl.ds(base, TPW)], logits_v)

    def bmaxf(x):
        return jnp.full((L,), plsc.cummax(x)[L - 1], jnp.float32)

    def bsumf(x):
        return jnp.full((L,), plsc.cumsum(x)[L - 1], jnp.float32)

    def bmini(x):
        return jnp.full((L,), -plsc.cummax(-x)[L - 1], jnp.int32)

    iotas = [lax.iota(jnp.int32, L) + L * j for j in range(E // L)]
    cnts = [jnp.zeros((L,), jnp.float32) for _ in range(E // L)]
    results = []
    for t in range(TPW):
        ls = [logits_v[t, pl.ds(L * j, L)] for j in range(E // L)]
        w1 = bmaxf(jnp.maximum(jnp.maximum(ls[0], ls[1]),
                               jnp.maximum(ls[2], ls[3])))
        c1 = [jnp.where(ls[j] >= w1, iotas[j], E) for j in range(4)]
        i1 = bmini(jnp.minimum(jnp.minimum(c1[0], c1[1]),
                               jnp.minimum(c1[2], c1[3])))
        ls2 = [jnp.where(iotas[j] == i1, -1e30, ls[j]) for j in range(4)]
        w2 = bmaxf(jnp.maximum(jnp.maximum(ls2[0], ls2[1]),
                               jnp.maximum(ls2[2], ls2[3])))
        c2 = [jnp.where(ls2[j] >= w2, iotas[j], E) for j in range(4)]
        i2 = bmini(jnp.minimum(jnp.minimum(c2[0], c2[1]),
                               jnp.minimum(c2[2], c2[3])))
        w1n = 1.0 / (1.0 + jnp.exp(w2 - w1))
        m1 = jnp.where(iotas[0] == i1, cnts[0], 0.0)
        for j in range(1, 4):
            m1 = m1 + jnp.where(iotas[j] == i1, cnts[j], 0.0)
        lpos1 = bsumf(m1)
        cnts = [cnts[j] + (iotas[j] == i1).astype(jnp.float32)
                for j in range(4)]
        m2 = jnp.where(iotas[0] == i2, cnts[0], 0.0)
        for j in range(1, 4):
            m2 = m2 + jnp.where(iotas[j] == i2, cnts[j], 0.0)
        lpos2 = bsumf(m2)
        cnts = [cnts[j] + (iotas[j] == i2).astype(jnp.float32)
                for j in range(4)]
        results.append((i1, i2, lpos1, lpos2, w1n))

    for j in range(4):
        cnt_v[pl.ds(L * j, L)] = cnts[j]
    pltpu.sync_copy(cnt_v, shared.at[pl.ds(wid * E, E)])
    plsc.subcore_barrier()
    pltpu.sync_copy(shared, all_v)
    pre = [jnp.zeros((L,), jnp.float32) for _ in range(4)]
    for w in range(NW):
        scale = (w < wid).astype(jnp.float32)
        for j in range(4):
            pre[j] = pre[j] + all_v[pl.ds(w * E + L * j, L)] * scale

    lane = lax.iota(jnp.int32, L)
    for t in range(TPW):
        i1, i2, lpos1, lpos2, w1n = results[t]
        g1 = jnp.where(iotas[0] == i1, pre[0], 0.0)
        g2 = jnp.where(iotas[0] == i2, pre[0], 0.0)
        for j in range(1, 4):
            g1 = g1 + jnp.where(iotas[j] == i1, pre[j], 0.0)
            g2 = g2 + jnp.where(iotas[j] == i2, pre[j], 0.0)
        p1 = lpos1 + bsumf(g1)
        p2 = lpos2 + bsumf(g2)
        p1 = jnp.where(p1 < CAP, p1, 255.0)
        p2 = jnp.where(p2 < CAP, p2, 255.0)
        vals = [i1.astype(jnp.float32), i2.astype(jnp.float32),
                p1, p2, w1n, 1.0 - w1n]
        row = jnp.zeros((L,), jnp.float32)
        for k, v in enumerate(vals):
            row = jnp.where(lane == k, v, row)
        out_v[t, :] = row
    pltpu.sync_copy(out_v, table_hbm.at[pl.ds(base, TPW)])


def _route_table(router_logits):
    return pl.kernel(
        _sc_routing,
        out_type=jax.ShapeDtypeStruct((T, L), jnp.float32),
        mesh=plsc.VectorSubcoreMesh(core_axis_name="c", subcore_axis_name="s",
                                    num_cores=NC),
        compiler_params=pltpu.CompilerParams(needs_layout_passes=False),
        scratch_types=[
            pltpu.VMEM((TPW, E), jnp.float32),
            pltpu.VMEM((TPW, L), jnp.float32),
            pltpu.VMEM((E,), jnp.float32),
            pltpu.VMEM((NW * E,), jnp.float32),
            pltpu.VMEM_SHARED((NW * E,), jnp.float32),
        ],
    )(router_logits)


def _moe_kernel(hid_ref, tab_ref, gu_ref, dp_ref, out_ref):
    e = pl.program_id(0)
    ef = e.astype(jnp.float32)
    i1f = tab_ref[:, 0:1]
    i2f = tab_ref[:, 1:2]
    pos1f = tab_ref[:, 2:3]
    pos2f = tab_ref[:, 3:4]
    w1f = tab_ref[:, 4:5]
    w2f = tab_ref[:, 5:6]
    capcol = jax.lax.broadcasted_iota(jnp.int32, (T, CAP), 1).astype(jnp.float32)
    sel1 = ((i1f == ef) & (pos1f == capcol)).astype(jnp.float32)
    sel2 = ((i2f == ef) & (pos2f == capcol)).astype(jnp.float32)
    sel = sel1 + sel2
    selw = sel1 * w1f + sel2 * w2f

    hid = hid_ref[...]
    xe = jax.lax.dot_general(sel, hid, (((0,), (0,)), ((), ())),
                             preferred_element_type=jnp.float32)
    gu = jnp.dot(xe.astype(jnp.bfloat16), gu_ref[0].astype(jnp.bfloat16),
                 preferred_element_type=jnp.float32)
    gate = jnp.minimum(gu[:, :FF], LIMIT)
    up = jnp.clip(gu[:, FF:], -LIMIT, LIMIT)
    glu = gate * jax.nn.sigmoid(gate * ALPHA)
    act = (up + 1.0) * glu
    out_b = jnp.dot(act.astype(jnp.bfloat16), dp_ref[0].astype(jnp.bfloat16),
                    preferred_element_type=jnp.float32)
    contrib = jnp.dot(selw, out_b, preferred_element_type=jnp.float32)

    @pl.when(e == 0)
    def _init():
        out_ref[...] = contrib

    @pl.when(e > 0)
    def _acc():
        out_ref[...] += contrib


def kernel(hidden_states, router_logits, gate_up_proj, down_proj):
    table = _route_table(router_logits)
    return pl.pallas_call(
        _moe_kernel,
        grid=(E,),
        in_specs=[
            pl.BlockSpec((T, D), lambda e: (0, 0)),
            pl.BlockSpec((T, L), lambda e: (0, 0)),
            pl.BlockSpec((1, D, 2 * FF), lambda e: (e, 0, 0)),
            pl.BlockSpec((1, FF, D), lambda e: (e, 0, 0)),
        ],
        out_specs=pl.BlockSpec((T, D), lambda e: (0, 0)),
        out_shape=jax.ShapeDtypeStruct((T, D), jnp.float32),
        compiler_params=pltpu.CompilerParams(
            dimension_semantics=("arbitrary",),
        ),
    )(hidden_states, table, gate_up_proj, down_proj)
